# Initial kernel scaffold; baseline (speedup 1.0000x reference)
import jax
import jax.numpy as jnp
from jax.experimental import pallas as pl


def kernel(values, table):
    raise NotImplementedError("write your pallas kernel here")


# speedup vs baseline: 5.7244x; 5.7244x over previous
import functools

import jax
import jax.numpy as jnp
from jax import lax
from jax.experimental import pallas as pl
from jax.experimental.pallas import tpu as pltpu
from jax.experimental.pallas import tpu_sc as plsc

_NBINS = 1000
_EMB = 32
_CHUNK = 128
_G = 4


def _bin_body(vals_ref, ids_ref):
    v = vals_ref[...]
    vmin = jnp.min(v, axis=0, keepdims=True)
    vmax = jnp.max(v, axis=0, keepdims=True)
    lo = jnp.log1p(vmin)
    hi = jnp.log1p(vmax)
    x = (jnp.log1p(v) - lo) / (hi - lo)
    x = jnp.clip(x, 0.0, 1.0)
    ids = (x * float(_NBINS)).astype(jnp.int32)
    ids_ref[...] = jnp.minimum(ids, _NBINS - 1)


def _compute_ids(values):
    return pl.pallas_call(
        _bin_body,
        out_shape=jax.ShapeDtypeStruct(values.shape, jnp.int32),
    )(values)


def _make_sc_gather(n_rows):
    mesh = plsc.VectorSubcoreMesh(core_axis_name="c", subcore_axis_name="s")
    nc = mesh.num_cores
    nw = nc * mesh.num_subcores
    n_chunk = n_rows // (nw * _CHUNK)
    assert n_chunk * nw * _CHUNK == n_rows and n_chunk % (2 * _G) == 0
    wave = _G * _CHUNK
    n_wave = n_chunk // _G
    n_pair = n_wave // 2
    b_per_w = n_chunk * _CHUNK

    @functools.partial(
        pl.kernel,
        out_type=jax.ShapeDtypeStruct((n_rows, _EMB), jnp.float32),
        mesh=mesh,
        scratch_types=[
            pltpu.VMEM((n_chunk, _CHUNK), jnp.int32),
            pltpu.VMEM((wave, _EMB), jnp.float32),
            pltpu.VMEM((wave, _EMB), jnp.float32),
            pltpu.SemaphoreType.DMA,
            pltpu.SemaphoreType.DMA,
            pltpu.SemaphoreType.DMA,
            pltpu.SemaphoreType.DMA,
        ],
        compiler_params=pltpu.CompilerParams(use_tc_tiling_on_sc=False),
    )
    def sc_gather(table_hbm, idx_hbm, out_hbm, idx_v, buf_a, buf_b,
                  gsem_a, gsem_b, osem_a, osem_b):
        wid = lax.axis_index("s") * nc + lax.axis_index("c")
        base = wid * b_per_w
        pltpu.sync_copy(idx_hbm.at[wid], idx_v)

        def issue_gathers(w, buf, gsem):
            for b in range(_G):
                pltpu.async_copy(
                    table_hbm.at[idx_v.at[w * _G + b]],
                    buf.at[pl.ds(b * _CHUNK, _CHUNK)], gsem)

        def drain_gathers(buf, gsem):
            for b in range(_G):
                pltpu.make_async_copy(
                    table_hbm.at[idx_v.at[0]],
                    buf.at[pl.ds(b * _CHUNK, _CHUNK)], gsem).wait()

        def issue_out(w, buf, osem):
            pltpu.async_copy(buf, out_hbm.at[pl.ds(base + w * wave, wave)], osem)

        def drain_out(buf, osem):
            pltpu.make_async_copy(buf, out_hbm.at[pl.ds(base, wave)], osem).wait()

        issue_gathers(0, buf_a, gsem_a)
        issue_gathers(1, buf_b, gsem_b)
        drain_gathers(buf_a, gsem_a)
        issue_out(0, buf_a, osem_a)
        drain_out(buf_a, osem_a)
        issue_gathers(2, buf_a, gsem_a)
        drain_gathers(buf_b, gsem_b)
        issue_out(1, buf_b, osem_b)

        def pair(p, carry):
            drain_out(buf_b, osem_b)
            issue_gathers(2 * p + 1, buf_b, gsem_b)
            drain_gathers(buf_a, gsem_a)
            issue_out(2 * p, buf_a, osem_a)
            drain_out(buf_a, osem_a)
            issue_gathers(2 * p + 2, buf_a, gsem_a)
            drain_gathers(buf_b, gsem_b)
            issue_out(2 * p + 1, buf_b, osem_b)
            return carry

        lax.fori_loop(1, n_pair - 1, pair, 0)

        w0 = 2 * (n_pair - 1)
        drain_out(buf_b, osem_b)
        issue_gathers(w0 + 1, buf_b, gsem_b)
        drain_gathers(buf_a, gsem_a)
        issue_out(w0, buf_a, osem_a)
        drain_out(buf_a, osem_a)
        drain_gathers(buf_b, gsem_b)
        issue_out(w0 + 1, buf_b, osem_b)
        drain_out(buf_b, osem_b)

    return sc_gather, nw


def kernel(values, table):
    batch, n_cols = values.shape
    n_rows = batch * n_cols
    ids = _compute_ids(values)
    sc_gather, nw = _make_sc_gather(n_rows)
    ids3 = ids.reshape(nw, -1, _CHUNK)
    out = sc_gather(table, ids3)
    return out.reshape(batch, n_cols, _EMB)


# speedup vs baseline: 6.1563x; 1.0755x over previous
import functools

import jax
import jax.numpy as jnp
from jax import lax
from jax.experimental import pallas as pl
from jax.experimental.pallas import tpu as pltpu
from jax.experimental.pallas import tpu_sc as plsc

_NBINS = 1000
_EMB = 32
_LANES = 16


def _binT_body(vals_ref, ids_ref):
    v = vals_ref[...]
    vmin = jnp.min(v, axis=1, keepdims=True)
    vmax = jnp.max(v, axis=1, keepdims=True)
    lo = jnp.log1p(vmin)
    hi = jnp.log1p(vmax)
    x = (jnp.log1p(v) - lo) / (hi - lo)
    x = jnp.clip(x, 0.0, 1.0)
    ids_ref[...] = (x * float(_NBINS)).astype(jnp.int32)


def _compute_idsT(vals_t):
    return pl.pallas_call(
        _binT_body,
        out_shape=jax.ShapeDtypeStruct(vals_t.shape, jnp.int32),
    )(vals_t)


def _make_sc_gather(n_cols, batch):
    mesh = plsc.VectorSubcoreMesh(core_axis_name="c", subcore_axis_name="s")
    nc = mesh.num_cores
    nw = nc * mesh.num_subcores
    bpw = batch // nw
    assert bpw * nw == batch and bpw % 128 == 0
    slab = n_cols * bpw
    tbl_flat = (_NBINS + 1) * _EMB
    groups = bpw // _LANES

    @functools.partial(
        pl.kernel,
        out_type=jax.ShapeDtypeStruct((n_cols, _EMB, batch), jnp.float32),
        mesh=mesh,
        scratch_types=[
            pltpu.VMEM((tbl_flat,), jnp.float32),
            pltpu.VMEM((slab,), jnp.int32),
            pltpu.VMEM((_EMB, bpw), jnp.float32),
            pltpu.VMEM((_EMB, bpw), jnp.float32),
            pltpu.SemaphoreType.DMA,
            pltpu.SemaphoreType.DMA,
        ],
        compiler_params=pltpu.CompilerParams(needs_layout_passes=False),
    )
    def sc_gather(tbl_hbm, ids_hbm, out_hbm, tbl_v, idx_v, buf_a, buf_b,
                  osem_a, osem_b):
        wid = lax.axis_index("s") * nc + lax.axis_index("c")
        i0 = wid * bpw
        pltpu.sync_copy(tbl_hbm, tbl_v)
        pltpu.sync_copy(ids_hbm.at[pl.ds(wid * slab, slab)], idx_v)

        def fill(c, buf):
            def g_body(g, carry):
                base = idx_v[pl.ds(c * bpw + _LANES * g, _LANES)] * _EMB
                for d in range(_EMB):
                    buf[d, pl.ds(_LANES * g, _LANES)] = plsc.load_gather(
                        tbl_v, [base + d])
                return carry
            lax.fori_loop(0, groups, g_body, 0)

        def issue_out(c, buf, osem):
            pltpu.async_copy(buf, out_hbm.at[c, :, pl.ds(i0, bpw)], osem)

        def drain_out(buf, osem):
            pltpu.make_async_copy(buf, out_hbm.at[0, :, pl.ds(i0, bpw)],
                                  osem).wait()

        fill(0, buf_a)
        issue_out(0, buf_a, osem_a)
        fill(1, buf_b)
        issue_out(1, buf_b, osem_b)

        def pair(p, carry):
            c = 2 * p
            drain_out(buf_a, osem_a)
            fill(c, buf_a)
            issue_out(c, buf_a, osem_a)
            drain_out(buf_b, osem_b)
            fill(c + 1, buf_b)
            issue_out(c + 1, buf_b, osem_b)
            return carry

        lax.fori_loop(1, n_cols // 2, pair, 0)
        drain_out(buf_a, osem_a)
        drain_out(buf_b, osem_b)

    return sc_gather, nw, bpw


def kernel(values, table):
    batch, n_cols = values.shape
    vals_t = jnp.transpose(values)
    ids_t = _compute_idsT(vals_t)
    sc_gather, nw, bpw = _make_sc_gather(n_cols, batch)
    ids_flat = jnp.transpose(
        ids_t.reshape(n_cols, nw, bpw), (1, 0, 2)).reshape(-1)
    table_ext = jnp.concatenate(
        [table, jnp.full((1, table.shape[1]), jnp.nan, table.dtype)], axis=0)
    tbl_flat = table_ext.reshape(-1)
    out = sc_gather(tbl_flat, ids_flat)
    return jnp.transpose(out, (2, 0, 1))


# speedup vs baseline: 9.9789x; 1.6209x over previous
import functools

import jax
import jax.numpy as jnp
from jax import lax
from jax.experimental import pallas as pl
from jax.experimental.pallas import tpu as pltpu
from jax.experimental.pallas import tpu_sc as plsc

_NBINS = 1000
_EMB = 32
_LANES = 16


def _binT_body(vals_ref, ids_ref):
    v = vals_ref[...]
    vmin = jnp.min(v, axis=1, keepdims=True)
    vmax = jnp.max(v, axis=1, keepdims=True)
    lo = jnp.log1p(vmin)
    hi = jnp.log1p(vmax)
    x = (jnp.log1p(v) - lo) / (hi - lo)
    x = jnp.clip(x, 0.0, 1.0)
    ids_ref[...] = (x * float(_NBINS)).astype(jnp.int32)


def _compute_idsT(vals_t):
    return pl.pallas_call(
        _binT_body,
        out_shape=jax.ShapeDtypeStruct(vals_t.shape, jnp.int32),
    )(vals_t)


def _make_sc_gather(n_cols, batch):
    mesh = plsc.VectorSubcoreMesh(core_axis_name="c", subcore_axis_name="s")
    nc = mesh.num_cores
    nw = nc * mesh.num_subcores
    bpw = batch // nw
    assert bpw * nw == batch and bpw % 128 == 0
    slab = n_cols * bpw
    tbl_flat = (_NBINS + 1) * _EMB
    groups = bpw // _LANES

    @functools.partial(
        pl.kernel,
        out_type=jax.ShapeDtypeStruct((n_cols, _EMB, batch), jnp.float32),
        mesh=mesh,
        scratch_types=[
            pltpu.VMEM((tbl_flat,), jnp.float32),
            pltpu.VMEM((slab,), jnp.int32),
            pltpu.VMEM((_EMB, bpw), jnp.float32),
            pltpu.VMEM((_EMB, bpw), jnp.float32),
            pltpu.SemaphoreType.DMA,
            pltpu.SemaphoreType.DMA,
        ],
        compiler_params=pltpu.CompilerParams(needs_layout_passes=False),
    )
    def sc_gather(tbl_hbm, ids_hbm, out_hbm, tbl_v, idx_v, buf_a, buf_b,
                  osem_a, osem_b):
        wid = lax.axis_index("s") * nc + lax.axis_index("c")
        i0 = wid * bpw
        pltpu.sync_copy(tbl_hbm, tbl_v)
        pltpu.sync_copy(ids_hbm.at[pl.ds(wid * slab, slab)], idx_v)

        def fill(c, buf):
            @plsc.parallel_loop(0, groups, 1, unroll=2)
            def g_body(g):
                base = idx_v[pl.ds(c * bpw + _LANES * g, _LANES)] * _EMB
                for d in range(_EMB):
                    buf[d, pl.ds(_LANES * g, _LANES)] = plsc.load_gather(
                        tbl_v, [base + d])

        def issue_out(c, buf, osem):
            pltpu.async_copy(buf, out_hbm.at[c, :, pl.ds(i0, bpw)], osem)

        def drain_out(buf, osem):
            pltpu.make_async_copy(buf, out_hbm.at[0, :, pl.ds(i0, bpw)],
                                  osem).wait()

        fill(0, buf_a)
        issue_out(0, buf_a, osem_a)
        fill(1, buf_b)
        issue_out(1, buf_b, osem_b)

        def pair(p, carry):
            c = 2 * p
            drain_out(buf_a, osem_a)
            fill(c, buf_a)
            issue_out(c, buf_a, osem_a)
            drain_out(buf_b, osem_b)
            fill(c + 1, buf_b)
            issue_out(c + 1, buf_b, osem_b)
            return carry

        lax.fori_loop(1, n_cols // 2, pair, 0)
        drain_out(buf_a, osem_a)
        drain_out(buf_b, osem_b)

    return sc_gather, nw, bpw


def kernel(values, table):
    batch, n_cols = values.shape
    vals_t = jnp.transpose(values)
    ids_t = _compute_idsT(vals_t)
    sc_gather, nw, bpw = _make_sc_gather(n_cols, batch)
    ids_flat = jnp.transpose(
        ids_t.reshape(n_cols, nw, bpw), (1, 0, 2)).reshape(-1)
    table_ext = jnp.concatenate(
        [table, jnp.full((1, table.shape[1]), jnp.nan, table.dtype)], axis=0)
    tbl_flat = table_ext.reshape(-1)
    out = sc_gather(tbl_flat, ids_flat)
    return jnp.transpose(out, (2, 0, 1))
